# baseline (speedup 1.0000x reference)
import jax
import jax.numpy as jnp
from jax import lax
from jax.experimental import pallas as pl
from jax.experimental.pallas import tpu as pltpu
from jax.experimental.pallas import tpu_sc as plsc

_TX = 8
_NS = 16
_NW = 32
_ROWS_PER_W = 32
_ZCH = 16
_ZL = 128


def _sc_rows_body(t_rows, p_rows, ridx_h, rows_o, ridx_v, rowbuf, sem):
    w = lax.axis_index("c") * _NS + lax.axis_index("s")
    pltpu.sync_copy(ridx_h.at[w], ridx_v)
    for c, table in ((0, t_rows), (1, p_rows)):
        pltpu.async_copy(table.at[ridx_v], rowbuf, sem).wait()
        pltpu.sync_copy(
            rowbuf, rows_o.at[c, pl.ds(w * _ROWS_PER_W, _ROWS_PER_W)])


def _sc_z_body(t_flat, p_flat, zidx_h, ozidx_h, out_ref,
               zidx_v, ozidx_v, zbuf, sem):
    w = lax.axis_index("c") * _NS + lax.axis_index("s")
    pltpu.sync_copy(zidx_h.at[w], zidx_v)
    for c, flat in ((0, t_flat), (1, p_flat)):
        hs = [
            pltpu.async_copy(flat.at[zidx_v.at[g]], zbuf.at[g], sem)
            for g in range(_ZCH)
        ]
        for h in hs:
            h.wait()
        pltpu.sync_copy(ozidx_h.at[c, w], ozidx_v)
        hs = [
            pltpu.async_copy(zbuf.at[g], out_ref.at[ozidx_v.at[g]], sem)
            for g in range(_ZCH)
        ]
        for h in hs:
            h.wait()


def _compose_body(coords_ref, pm_ref, rows_ref, out_ref):
    b = pl.program_id(0)
    xt = pl.program_id(1)
    cx = coords_ref[b, 0]
    cy = coords_ref[b, 1]

    shape = (_TX, pm_ref.shape[3], pm_ref.shape[4])
    x_ids = jax.lax.broadcasted_iota(jnp.int32, shape, 0) + xt * _TX
    y_ids = jax.lax.broadcasted_iota(jnp.int32, shape, 1)

    out_ref[0, 2] = pm_ref[0, 0]
    for c in range(2):
        xsl = rows_ref[c, 0, 0]
        yr = rows_ref[c, 0, 1, pl.ds(xt * _TX, _TX)]
        val = jnp.where(x_ids == cx, xsl[None],
                        jnp.where(y_ids == cy, yr[:, None, :], 0.0))
        out_ref[0, c] = val


def kernel(full_target_img, full_prior_img, prior_mask, coords):
    B, C, W, H, D = full_target_img.shape
    nxt = W // _TX
    f32 = full_target_img.dtype
    coords = coords.astype(jnp.int32)
    cx, cy, cz = coords[:, 0], coords[:, 1], coords[:, 2]

    k = jnp.arange(B * 2 * H, dtype=jnp.int32)
    kb = k // (2 * H)
    kr = k % (2 * H)
    ridx = jnp.where(
        kr < H,
        (kb * W + cx[kb]) * H + kr,
        (kb * W + (kr - H)) * H + cy[kb],
    ).reshape(_NW, _ROWS_PER_W)

    e = jnp.arange(B * W * H, dtype=jnp.int32)
    eb = e // (W * H)
    er = e % (W * H)
    ex = er // H
    ey = er % H
    zidx = ((((eb * W + ex) * H + ey) * D) + cz[eb]).reshape(_NW, _ZCH, _ZL)
    ozidx = jnp.stack([
        ((((eb * 3 + c) * W + ex) * H + ey) * D + cz[eb]).reshape(
            _NW, _ZCH, _ZL)
        for c in range(2)
    ])

    t_rows = full_target_img.reshape(B * W * H, D)
    p_rows = full_prior_img.reshape(B * W * H, D)
    t_flat = full_target_img.reshape(-1)
    p_flat = full_prior_img.reshape(-1)

    mesh = plsc.VectorSubcoreMesh(core_axis_name="c", subcore_axis_name="s")

    rows = pl.kernel(
        _sc_rows_body,
        out_type=jax.ShapeDtypeStruct((2, B * 2 * H, D), f32),
        mesh=mesh,
        scratch_types=[
            pltpu.VMEM((_ROWS_PER_W,), jnp.int32),
            pltpu.VMEM((_ROWS_PER_W, D), f32),
            pltpu.SemaphoreType.DMA,
        ],
    )(t_rows, p_rows, ridx)
    rows5 = rows.reshape(2, B, 2, H, D)

    grid_spec = pltpu.PrefetchScalarGridSpec(
        num_scalar_prefetch=1,
        grid=(B, nxt),
        in_specs=[
            pl.BlockSpec((1, 1, _TX, H, D), lambda b, xt, c_ref: (b, 0, xt, 0, 0)),
            pl.BlockSpec((2, 1, 2, H, D), lambda b, xt, c_ref: (0, b, 0, 0, 0)),
        ],
        out_specs=pl.BlockSpec((1, 3, _TX, H, D),
                               lambda b, xt, c_ref: (b, 0, xt, 0, 0)),
    )
    out = pl.pallas_call(
        _compose_body,
        grid_spec=grid_spec,
        out_shape=jax.ShapeDtypeStruct((B, 3, W, H, D), f32),
    )(coords, prior_mask, rows5)

    z_scatter = pl.kernel(
        _sc_z_body,
        out_type=(),
        mesh=mesh,
        scratch_types=[
            pltpu.VMEM((_ZCH, _ZL), jnp.int32),
            pltpu.VMEM((_ZCH, _ZL), jnp.int32),
            pltpu.VMEM((_ZCH, _ZL), f32),
            pltpu.SemaphoreType.DMA,
        ],
    )
    out_ref = jax.new_ref(out.reshape(-1))
    z_scatter(t_flat, p_flat, zidx, ozidx, out_ref)
    return out_ref[...].reshape(B, 3, W, H, D)


# speedup vs baseline: 3.4024x; 3.4024x over previous
import jax
import jax.numpy as jnp
from jax.experimental import pallas as pl
from jax.experimental.pallas import tpu as pltpu

_TX = 16


def _body(coords_ref, target_ref, prior_ref, mask_ref, out_ref):
    b = pl.program_id(0)
    xt = pl.program_id(1)
    cx = coords_ref[b, 0]
    cy = coords_ref[b, 1]
    cz = coords_ref[b, 2]

    shape = target_ref.shape[2:]
    x_ids = jax.lax.broadcasted_iota(jnp.int32, shape, 0) + xt * _TX
    y_ids = jax.lax.broadcasted_iota(jnp.int32, shape, 1)
    z_ids = jax.lax.broadcasted_iota(jnp.int32, shape, 2)
    m = (x_ids == cx) | (y_ids == cy) | (z_ids == cz)

    zero = jnp.zeros(shape, dtype=out_ref.dtype)
    out_ref[0, 0] = jnp.where(m, target_ref[0, 0], zero)
    out_ref[0, 1] = jnp.where(m, prior_ref[0, 0], zero)
    out_ref[0, 2] = mask_ref[0, 0]


def kernel(full_target_img, full_prior_img, prior_mask, coords):
    B, C, W, H, D = full_target_img.shape
    nxt = W // _TX

    def in_map(b, xt, coords_ref):
        return (b, 0, xt, 0, 0)

    def out_map(b, xt, coords_ref):
        return (b, 0, xt, 0, 0)

    grid_spec = pltpu.PrefetchScalarGridSpec(
        num_scalar_prefetch=1,
        grid=(B, nxt),
        in_specs=[
            pl.BlockSpec((1, 1, _TX, H, D), in_map),
            pl.BlockSpec((1, 1, _TX, H, D), in_map),
            pl.BlockSpec((1, 1, _TX, H, D), in_map),
        ],
        out_specs=pl.BlockSpec((1, 3, _TX, H, D), out_map),
    )

    return pl.pallas_call(
        _body,
        grid_spec=grid_spec,
        out_shape=jax.ShapeDtypeStruct((B, 3, W, H, D), full_target_img.dtype),
    )(coords, full_target_img, full_prior_img, prior_mask)


# speedup vs baseline: 3.5515x; 1.0438x over previous
import jax
import jax.numpy as jnp
from jax.experimental import pallas as pl
from jax.experimental.pallas import tpu as pltpu

_TX = 32


def _body(coords_ref, target_ref, prior_ref, mask_ref, out_ref):
    b = pl.program_id(0)
    xt = pl.program_id(1)
    cx = coords_ref[b, 0]
    cy = coords_ref[b, 1]
    cz = coords_ref[b, 2]

    shape = target_ref.shape[2:]
    x_ids = jax.lax.broadcasted_iota(jnp.int32, shape, 0) + xt * _TX
    y_ids = jax.lax.broadcasted_iota(jnp.int32, shape, 1)
    z_ids = jax.lax.broadcasted_iota(jnp.int32, shape, 2)
    m = (x_ids == cx) | (y_ids == cy) | (z_ids == cz)

    zero = jnp.zeros(shape, dtype=out_ref.dtype)
    out_ref[0, 0] = jnp.where(m, target_ref[0, 0], zero)
    out_ref[0, 1] = jnp.where(m, prior_ref[0, 0], zero)
    out_ref[0, 2] = mask_ref[0, 0]


def kernel(full_target_img, full_prior_img, prior_mask, coords):
    B, C, W, H, D = full_target_img.shape
    nxt = W // _TX

    def in_map(b, xt, coords_ref):
        return (b, 0, xt, 0, 0)

    def out_map(b, xt, coords_ref):
        return (b, 0, xt, 0, 0)

    grid_spec = pltpu.PrefetchScalarGridSpec(
        num_scalar_prefetch=1,
        grid=(B, nxt),
        in_specs=[
            pl.BlockSpec((1, 1, _TX, H, D), in_map),
            pl.BlockSpec((1, 1, _TX, H, D), in_map),
            pl.BlockSpec((1, 1, _TX, H, D), in_map),
        ],
        out_specs=pl.BlockSpec((1, 3, _TX, H, D), out_map),
    )

    return pl.pallas_call(
        _body,
        grid_spec=grid_spec,
        out_shape=jax.ShapeDtypeStruct((B, 3, W, H, D), full_target_img.dtype),
    )(coords, full_target_img, full_prior_img, prior_mask)


# speedup vs baseline: 3.6188x; 1.0190x over previous
import jax
import jax.numpy as jnp
from jax.experimental import pallas as pl
from jax.experimental.pallas import tpu as pltpu

_TX = 64


def _body(coords_ref, target_ref, prior_ref, mask_ref, out_ref):
    b = pl.program_id(0)
    xt = pl.program_id(1)
    cx = coords_ref[b, 0]
    cy = coords_ref[b, 1]
    cz = coords_ref[b, 2]

    shape = target_ref.shape[2:]
    x_ids = jax.lax.broadcasted_iota(jnp.int32, shape, 0) + xt * _TX
    y_ids = jax.lax.broadcasted_iota(jnp.int32, shape, 1)
    z_ids = jax.lax.broadcasted_iota(jnp.int32, shape, 2)
    m = (x_ids == cx) | (y_ids == cy) | (z_ids == cz)

    zero = jnp.zeros(shape, dtype=out_ref.dtype)
    out_ref[0, 0] = jnp.where(m, target_ref[0, 0], zero)
    out_ref[0, 1] = jnp.where(m, prior_ref[0, 0], zero)
    out_ref[0, 2] = mask_ref[0, 0]


def kernel(full_target_img, full_prior_img, prior_mask, coords):
    B, C, W, H, D = full_target_img.shape
    nxt = W // _TX

    def in_map(b, xt, coords_ref):
        return (b, 0, xt, 0, 0)

    def out_map(b, xt, coords_ref):
        return (b, 0, xt, 0, 0)

    grid_spec = pltpu.PrefetchScalarGridSpec(
        num_scalar_prefetch=1,
        grid=(B, nxt),
        in_specs=[
            pl.BlockSpec((1, 1, _TX, H, D), in_map),
            pl.BlockSpec((1, 1, _TX, H, D), in_map),
            pl.BlockSpec((1, 1, _TX, H, D), in_map),
        ],
        out_specs=pl.BlockSpec((1, 3, _TX, H, D), out_map),
    )

    return pl.pallas_call(
        _body,
        grid_spec=grid_spec,
        out_shape=jax.ShapeDtypeStruct((B, 3, W, H, D), full_target_img.dtype),
    )(coords, full_target_img, full_prior_img, prior_mask)


# speedup vs baseline: 3.6648x; 1.0127x over previous
import jax
import jax.numpy as jnp
from jax.experimental import pallas as pl
from jax.experimental.pallas import tpu as pltpu

_TX = 64


def _body(coords_ref, target_ref, prior_ref, mask_ref, out_ref):
    b = pl.program_id(0)
    xt = pl.program_id(1)
    cx = coords_ref[b, 0]
    cy = coords_ref[b, 1]
    cz = coords_ref[b, 2]

    shape = target_ref.shape[2:]
    y_ids = jax.lax.broadcasted_iota(jnp.int32, (1,) + shape[1:], 1)
    z_ids = jax.lax.broadcasted_iota(jnp.int32, (1,) + shape[1:], 2)
    m_yz = (y_ids == cy) | (z_ids == cz)

    zero = jnp.zeros(shape, dtype=out_ref.dtype)
    out_ref[0, 0] = jnp.where(m_yz, target_ref[0, 0], zero)
    out_ref[0, 1] = jnp.where(m_yz, prior_ref[0, 0], zero)
    out_ref[0, 2] = mask_ref[0, 0]

    lcx = cx - xt * _TX

    @pl.when((lcx >= 0) & (lcx < _TX))
    def _():
        out_ref[0, 0, pl.ds(lcx, 1)] = target_ref[0, 0, pl.ds(lcx, 1)]
        out_ref[0, 1, pl.ds(lcx, 1)] = prior_ref[0, 0, pl.ds(lcx, 1)]


def kernel(full_target_img, full_prior_img, prior_mask, coords):
    B, C, W, H, D = full_target_img.shape
    nxt = W // _TX

    def in_map(b, xt, coords_ref):
        return (b, 0, xt, 0, 0)

    def out_map(b, xt, coords_ref):
        return (b, 0, xt, 0, 0)

    grid_spec = pltpu.PrefetchScalarGridSpec(
        num_scalar_prefetch=1,
        grid=(B, nxt),
        in_specs=[
            pl.BlockSpec((1, 1, _TX, H, D), in_map),
            pl.BlockSpec((1, 1, _TX, H, D), in_map),
            pl.BlockSpec((1, 1, _TX, H, D), in_map),
        ],
        out_specs=pl.BlockSpec((1, 3, _TX, H, D), out_map),
    )

    return pl.pallas_call(
        _body,
        grid_spec=grid_spec,
        out_shape=jax.ShapeDtypeStruct((B, 3, W, H, D), full_target_img.dtype),
    )(coords, full_target_img, full_prior_img, prior_mask)
